# baseline (speedup 1.0000x reference)
import functools

import jax
import jax.numpy as jnp
from jax import lax
from jax.experimental import pallas as pl
from jax.experimental.pallas import tpu as pltpu

N = 10000
E = 2000
H = 256
BK = 400
NKB = N // BK


def _bf(x):
    return x.astype(jnp.bfloat16)


def _net_kernel(inc_ref, x_ref, w10_ref, w20_ref, w11_ref, w21_ref,
                xout_ref, x1out_ref, incs_ref, xw1s_ref, xw2s_ref, acc_ref):
    p = pl.program_id(0)
    i = pl.program_id(1)
    rows = pl.ds(i * BK, BK)

    @pl.when(p == 0)
    def _phase0():
        inc_blk = _bf(inc_ref[...])
        incs_ref[rows, :] = inc_blk

        @pl.when(i == 0)
        def _():
            acc_ref[...] = jnp.zeros_like(acc_ref)

        xw1 = jnp.dot(x_ref[...], w10_ref[...],
                      preferred_element_type=jnp.float32)
        acc_ref[...] += lax.dot_general(
            inc_blk, _bf(xw1), (((0,), (0,)), ((), ())),
            preferred_element_type=jnp.float32)

        @pl.when(i == NKB - 1)
        def _():
            x1 = jnp.maximum(acc_ref[...], 0.0)
            xw2s_ref[...] = _bf(jnp.dot(x1, w20_ref[...],
                                        preferred_element_type=jnp.float32))

    @pl.when(p == 1)
    def _phase1():
        t = jnp.maximum(
            jnp.dot(incs_ref[rows, :], xw2s_ref[...],
                    preferred_element_type=jnp.float32), 0.0)
        xw1s_ref[rows, :] = _bf(jnp.dot(t, w11_ref[...],
                                        preferred_element_type=jnp.float32))

    @pl.when(p == 2)
    def _phase2():
        @pl.when(i == 0)
        def _():
            acc_ref[...] = jnp.zeros_like(acc_ref)

        acc_ref[...] += lax.dot_general(
            incs_ref[rows, :], xw1s_ref[rows, :], (((0,), (0,)), ((), ())),
            preferred_element_type=jnp.float32)

        @pl.when(i == NKB - 1)
        def _():
            x1 = jnp.maximum(acc_ref[...], 0.0)
            x1out_ref[...] = x1
            xw2s_ref[...] = _bf(jnp.dot(x1, w21_ref[...],
                                        preferred_element_type=jnp.float32))

    @pl.when(p == 3)
    def _phase3():
        xout_ref[...] = jnp.maximum(
            jnp.dot(incs_ref[rows, :], xw2s_ref[...],
                    preferred_element_type=jnp.float32), 0.0)


def kernel(x_0, incidence_1, weight1_0, weight2_0, att_weight1_0, att_weight2_0,
           weight1_1, weight2_1, att_weight1_1, att_weight2_1):
    first = lambda p, i: (jnp.where(p == 0, i, 0), 0)
    last = lambda p, i: (jnp.where(p == 3, i, 0), 0)
    const = lambda p, i: (0, 0)
    x_out, x1_out = pl.pallas_call(
        _net_kernel,
        grid=(4, NKB),
        in_specs=[
            pl.BlockSpec((BK, E), first),
            pl.BlockSpec((BK, H), first),
            pl.BlockSpec((H, H), const),
            pl.BlockSpec((H, H), const),
            pl.BlockSpec((H, H), const),
            pl.BlockSpec((H, H), const),
        ],
        out_specs=[
            pl.BlockSpec((BK, H), last),
            pl.BlockSpec((E, H), const),
        ],
        out_shape=[
            jax.ShapeDtypeStruct((N, H), jnp.float32),
            jax.ShapeDtypeStruct((E, H), jnp.float32),
        ],
        scratch_shapes=[
            pltpu.VMEM((N, E), jnp.bfloat16),
            pltpu.VMEM((N, H), jnp.bfloat16),
            pltpu.VMEM((E, H), jnp.bfloat16),
            pltpu.VMEM((E, H), jnp.float32),
        ],
        compiler_params=pltpu.CompilerParams(
            vmem_limit_bytes=100 * 1024 * 1024),
    )(incidence_1, x_0, weight1_0, weight2_0, weight1_1, weight2_1)
    return (x_out, x1_out)
